# Initial kernel scaffold; baseline (speedup 1.0000x reference)
import jax
import jax.numpy as jnp
from jax.experimental import pallas as pl


def kernel(prediction, target):
    raise NotImplementedError("write your pallas kernel here")


# speedup vs baseline: 41.8660x; 41.8660x over previous
import functools

import jax
import jax.numpy as jnp
from jax import lax
from jax.experimental import pallas as pl
from jax.experimental.pallas import tpu as pltpu
from jax.experimental.pallas import tpu_sc as plsc

B = 4
F = 16
HW = 512 * 512
K = 32
DELTA_V = 0.5
DELTA_D = 1.5
ALPHA, BETA, GAMMA = 1.0, 1.0, 0.001

NC = 2
NS = 16
NW = NC * NS
TPS = NW // B
PPT = HW // TPS
BLK = 2048
NVREG = BLK // 16
NBLK = PPT // BLK
ACC1 = (F + 1) * K
P2W = 48

_mesh = plsc.VectorSubcoreMesh(core_axis_name="c", subcore_axis_name="s")


def _worker(c, s):
    return s * NC + c


def _pass1_body(pred_hbm, lab_hbm, out_hbm, pbuf, lbuf, acc, sem):
    w = _worker(lax.axis_index("c"), lax.axis_index("s"))
    b = w // TPS
    chunk = w % TPS
    zero = jnp.zeros((16,), jnp.float32)
    ones = jnp.ones((16,), jnp.float32)
    for i in range(ACC1 // 16):
        acc[pl.ds(i * 16, 16)] = zero

    def do_block(bi, carry):
        off = b * HW + chunk * PPT + bi * BLK
        copies = []
        for f in range(F):
            src = (b * F + f) * HW + chunk * PPT + bi * BLK
            copies.append(pltpu.async_copy(
                pred_hbm.at[pl.ds(src, BLK)], pbuf.at[pl.ds(f * BLK, BLK)], sem))
        lcpy = pltpu.async_copy(lab_hbm.at[pl.ds(off, BLK)], lbuf, sem)
        for cp in copies:
            cp.wait()
        lcpy.wait()

        def vbody(j, c2):
            lab = lbuf[pl.ds(j * 16, 16)]
            for f in range(F):
                vals = pbuf[pl.ds(f * BLK + j * 16, 16)]
                plsc.addupdate_scatter(acc, [lab + f * K], vals)
            plsc.addupdate_scatter(acc, [lab + F * K], ones)
            return c2

        return lax.fori_loop(0, NVREG, vbody, carry)

    lax.fori_loop(0, NBLK, do_block, 0)
    pltpu.sync_copy(acc, out_hbm.at[pl.ds(w * ACC1, ACC1)])


_sc_params = pltpu.CompilerParams(needs_layout_passes=False)

_pass1 = pl.kernel(
    _pass1_body,
    out_type=jax.ShapeDtypeStruct((NW * ACC1,), jnp.float32),
    mesh=_mesh,
    compiler_params=_sc_params,
    scratch_types=[
        pltpu.VMEM((F * BLK,), jnp.float32),
        pltpu.VMEM((BLK,), jnp.int32),
        pltpu.VMEM((ACC1,), jnp.float32),
        pltpu.SemaphoreType.DMA,
    ],
)


def _sqrt16(x):
    i = plsc.bitcast(x, jnp.int32)
    g = plsc.bitcast((i >> 1) + 0x1FBD1DF5, jnp.float32)
    y = 0.5 * (g + x / g)
    y = 0.5 * (y + x / y)
    y = 0.5 * (y + x / y)
    return y


def _pass2_body(pred_hbm, lab_hbm, p1_hbm, out_hbm, pbuf, lbuf, p1buf, mu, obuf, sem):
    w = _worker(lax.axis_index("c"), lax.axis_index("s"))
    b = w // TPS
    chunk = w % TPS
    zero = jnp.zeros((16,), jnp.float32)

    pltpu.sync_copy(p1_hbm.at[pl.ds(b * TPS * ACC1, TPS * ACC1)], p1buf)

    invc = []
    for half in range(2):
        c = p1buf[pl.ds(F * K + half * 16, 16)]
        for t in range(1, TPS):
            c = c + p1buf[pl.ds(t * ACC1 + F * K + half * 16, 16)]
        invc.append(1.0 / jnp.maximum(c, 1.0))
    for f in range(F):
        for half in range(2):
            s = p1buf[pl.ds(f * K + half * 16, 16)]
            for t in range(1, TPS):
                s = s + p1buf[pl.ds(t * ACC1 + f * K + half * 16, 16)]
            mu[pl.ds(f * K + half * 16, 16)] = s * invc[half]

    for i in range(P2W // 16):
        obuf[pl.ds(i * 16, 16)] = zero

    def do_block(bi, dmax):
        off = b * HW + chunk * PPT + bi * BLK
        copies = []
        for f in range(F):
            src = (b * F + f) * HW + chunk * PPT + bi * BLK
            copies.append(pltpu.async_copy(
                pred_hbm.at[pl.ds(src, BLK)], pbuf.at[pl.ds(f * BLK, BLK)], sem))
        lcpy = pltpu.async_copy(lab_hbm.at[pl.ds(off, BLK)], lbuf, sem)
        for cp in copies:
            cp.wait()
        lcpy.wait()

        def vbody(j, dmx):
            lab = lbuf[pl.ds(j * 16, 16)]
            d2 = jnp.zeros((16,), jnp.float32)
            for f in range(F):
                m = plsc.load_gather(mu, [lab + f * K])
                p = pbuf[pl.ds(f * BLK + j * 16, 16)]
                diff = m - p
                d2 = d2 + diff * diff
            d = _sqrt16(d2) - DELTA_V
            r = jnp.maximum(d, 0.0)
            plsc.addupdate_scatter(obuf, [lab], r * r)
            return jnp.maximum(dmx, d)

        return lax.fori_loop(0, NVREG, vbody, dmax)

    dmax = lax.fori_loop(0, NBLK, do_block,
                         jnp.full((16,), -3e38, jnp.float32))
    obuf[pl.ds(32, 16)] = dmax
    pltpu.sync_copy(obuf, out_hbm.at[pl.ds(w * P2W, P2W)])


_pass2 = pl.kernel(
    _pass2_body,
    out_type=jax.ShapeDtypeStruct((NW * P2W,), jnp.float32),
    mesh=_mesh,
    compiler_params=_sc_params,
    scratch_types=[
        pltpu.VMEM((F * BLK,), jnp.float32),
        pltpu.VMEM((BLK,), jnp.int32),
        pltpu.VMEM((TPS * ACC1,), jnp.float32),
        pltpu.VMEM((F * K,), jnp.float32),
        pltpu.VMEM((P2W,), jnp.float32),
        pltpu.SemaphoreType.DMA,
    ],
)


def _final_body(p1_ref, p2_ref, out_ref):
    p1 = p1_ref[...]
    s = jnp.sum(p1, axis=1)
    counts = s[:, F, :]
    counts_safe = jnp.maximum(counts, 1.0)
    mu = s[:, :F, :] / counts_safe[:, None, :]
    musq = jnp.sum(mu * mu, axis=1)
    l_reg = jnp.sum(jnp.sqrt(musq), axis=1) / K

    p2 = p2_ref[...]
    lsum = jnp.sum(p2[:, :, :K], axis=1)
    dmax = jnp.max(p2[:, :, K:], axis=(1, 2))
    l_var = jnp.sum(lsum / counts_safe, axis=1) / K
    nonempty = jnp.sum((counts > 0.0).astype(jnp.float32), axis=1)
    l_var = jnp.where(dmax < 0.0, dmax * dmax * nonempty / K, l_var)

    ci = lax.broadcasted_iota(jnp.int32, (1, B), 1)
    ld_vec = jnp.zeros((1, B), jnp.float32)
    eye_i = lax.broadcasted_iota(jnp.int32, (K, K), 0)
    eye_j = lax.broadcasted_iota(jnp.int32, (K, K), 1)
    off_diag = jnp.where(eye_i == eye_j, 0.0, 1.0)
    for bb in range(B):
        m = mu[bb]
        g = lax.dot_general(m, m, (((0,), (0,)), ((), ())))
        nsq = musq[bb]
        d2 = jnp.maximum(nsq[:, None] + nsq[None, :] - 2.0 * g, 0.0)
        d = jnp.sqrt(d2)
        lm = 2.0 * DELTA_D * off_diag - d
        r = jnp.maximum(lm, 0.0) * off_diag
        ld_b = jnp.sum(r * r) / K / (K - 1)
        ld_vec = ld_vec + ld_b * jnp.where(ci == bb, 1.0, 0.0)

    lv_vec = l_var[None, :]
    lr_vec = l_reg[None, :]
    loss_vec = ALPHA * lv_vec + BETA * ld_vec + GAMMA * lr_vec
    lm_ = jnp.sum(loss_vec) / B
    vm_ = jnp.sum(lv_vec) / B
    dm_ = jnp.sum(ld_vec) / B
    rm_ = jnp.sum(lr_vec) / B
    col = lax.broadcasted_iota(jnp.int32, (8, 128), 1)
    out_ref[...] = (jnp.where(col == 0, lm_, 0.0) + jnp.where(col == 1, vm_, 0.0)
                    + jnp.where(col == 2, dm_, 0.0) + jnp.where(col == 3, rm_, 0.0))


_finalize = pl.pallas_call(
    _final_body,
    out_shape=jax.ShapeDtypeStruct((8, 128), jnp.float32),
)


def kernel(prediction, target):
    pred_flat = prediction.reshape(-1)
    lab_flat = target.astype(jnp.int32).reshape(-1)
    p1 = _pass1(pred_flat, lab_flat)
    p2 = _pass2(pred_flat, lab_flat, p1)
    res = _finalize(p1.reshape(B, TPS, F + 1, K), p2.reshape(B, TPS, P2W))
    return (res[0, 0], res[0, 1], res[0, 2], res[0, 3])


# speedup vs baseline: 47.8933x; 1.1440x over previous
import functools

import jax
import jax.numpy as jnp
from jax import lax
from jax.experimental import pallas as pl
from jax.experimental.pallas import tpu as pltpu
from jax.experimental.pallas import tpu_sc as plsc

B = 4
F = 16
HW = 512 * 512
K = 32
DELTA_V = 0.5
DELTA_D = 1.5
ALPHA, BETA, GAMMA = 1.0, 1.0, 0.001

NC = 2
NS = 16
NW = NC * NS
TPS = NW // B
PPT = HW // TPS
BLK = 2048
NVREG = BLK // 16
NBLK = PPT // BLK
ACC1 = (F + 1) * K
P2W = 48

_mesh = plsc.VectorSubcoreMesh(core_axis_name="c", subcore_axis_name="s")
_sc_params = pltpu.CompilerParams(needs_layout_passes=False)


def _sqrt16(x):
    i = plsc.bitcast(x, jnp.int32)
    g = plsc.bitcast((i >> 1) + 0x1FBD1DF5, jnp.float32)
    y = 0.5 * (g + x / g)
    y = 0.5 * (y + x / y)
    y = 0.5 * (y + x / y)
    return y


def _issue_block(pred_hbm, lab_hbm, pbuf, lbuf, sem, b, chunk, bi, phase):
    for f in range(F):
        src = (b * F + f) * HW + chunk * PPT + bi * BLK
        pltpu.async_copy(pred_hbm.at[pl.ds(src, BLK)],
                         pbuf.at[pl.ds((phase * F + f) * BLK, BLK)], sem)
    off = b * HW + chunk * PPT + bi * BLK
    pltpu.async_copy(lab_hbm.at[pl.ds(off, BLK)],
                     lbuf.at[pl.ds(phase * BLK, BLK)], sem)


def _wait_block(pred_hbm, lab_hbm, pbuf, lbuf, sem, phase):
    pltpu.make_async_copy(pred_hbm.at[pl.ds(0, F * BLK)],
                          pbuf.at[pl.ds(phase * F * BLK, F * BLK)], sem).wait()
    pltpu.make_async_copy(lab_hbm.at[pl.ds(0, BLK)],
                          lbuf.at[pl.ds(phase * BLK, BLK)], sem).wait()


def _fused_body(pred_hbm, lab_hbm, p1_hbm, p2_hbm, pbuf, lbuf, acc, p1buf,
                mu, obuf, sem):
    w = lax.axis_index("c") * NS + lax.axis_index("s")
    b = w // TPS
    chunk = w % TPS
    zero = jnp.zeros((16,), jnp.float32)
    ones = jnp.ones((16,), jnp.float32)

    issue = functools.partial(_issue_block, pred_hbm, lab_hbm, pbuf, lbuf,
                              sem, b, chunk)
    wait = functools.partial(_wait_block, pred_hbm, lab_hbm, pbuf, lbuf, sem)

    for i in range(ACC1 // 16):
        acc[pl.ds(i * 16, 16)] = zero

    issue(0, 0)
    issue(1, 1)

    def p1_block(g, carry):
        for phase in range(2):
            bi = g * 2 + phase
            wait(phase)

            def vbody(jj, c2):
                for u in range(2):
                    j = jj * 2 + u
                    lab = lbuf[pl.ds(phase * BLK + j * 16, 16)]
                    for f in range(F):
                        vals = pbuf[pl.ds((phase * F + f) * BLK + j * 16, 16)]
                        plsc.addupdate_scatter(acc, [lab + f * K], vals)
                    plsc.addupdate_scatter(acc, [lab + F * K], ones)
                return c2

            lax.fori_loop(0, NVREG // 2, vbody, 0)
            issue((bi + 2) % NBLK, phase)
        return carry

    lax.fori_loop(0, NBLK // 2, p1_block, 0)
    wait(0)
    wait(1)

    pltpu.sync_copy(acc, p1_hbm.at[pl.ds(w * ACC1, ACC1)])
    plsc.subcore_barrier()

    pltpu.sync_copy(p1_hbm.at[pl.ds(b * TPS * ACC1, TPS * ACC1)], p1buf)
    invc = []
    for half in range(2):
        c = p1buf[pl.ds(F * K + half * 16, 16)]
        for t in range(1, TPS):
            c = c + p1buf[pl.ds(t * ACC1 + F * K + half * 16, 16)]
        invc.append(1.0 / jnp.maximum(c, 1.0))
    for f in range(F):
        for half in range(2):
            s = p1buf[pl.ds(f * K + half * 16, 16)]
            for t in range(1, TPS):
                s = s + p1buf[pl.ds(t * ACC1 + f * K + half * 16, 16)]
            mu[pl.ds(f * K + half * 16, 16)] = s * invc[half]

    for i in range(P2W // 16):
        obuf[pl.ds(i * 16, 16)] = zero

    issue(0, 0)
    issue(1, 1)

    def p2_block(g, dmax):
        for phase in range(2):
            bi = g * 2 + phase
            wait(phase)

            def vbody(jj, dmx):
                for u in range(2):
                    j = jj * 2 + u
                    lab = lbuf[pl.ds(phase * BLK + j * 16, 16)]
                    d2 = jnp.zeros((16,), jnp.float32)
                    for f in range(F):
                        m = plsc.load_gather(mu, [lab + f * K])
                        p = pbuf[pl.ds((phase * F + f) * BLK + j * 16, 16)]
                        diff = m - p
                        d2 = d2 + diff * diff
                    d = _sqrt16(d2) - DELTA_V
                    r = jnp.maximum(d, 0.0)
                    plsc.addupdate_scatter(obuf, [lab], r * r)
                    dmx = jnp.maximum(dmx, d)
                return dmx

            dmax = lax.fori_loop(0, NVREG // 2, vbody, dmax)
            issue((bi + 2) % NBLK, phase)
        return dmax

    dmax = lax.fori_loop(0, NBLK // 2, p2_block,
                         jnp.full((16,), -3e38, jnp.float32))
    wait(0)
    wait(1)

    obuf[pl.ds(32, 16)] = dmax
    pltpu.sync_copy(obuf, p2_hbm.at[pl.ds(w * P2W, P2W)])


_fused = pl.kernel(
    _fused_body,
    out_type=(jax.ShapeDtypeStruct((NW * ACC1,), jnp.float32),
              jax.ShapeDtypeStruct((NW * P2W,), jnp.float32)),
    mesh=_mesh,
    compiler_params=_sc_params,
    scratch_types=[
        pltpu.VMEM((2 * F * BLK,), jnp.float32),
        pltpu.VMEM((2 * BLK,), jnp.int32),
        pltpu.VMEM((ACC1,), jnp.float32),
        pltpu.VMEM((TPS * ACC1,), jnp.float32),
        pltpu.VMEM((F * K,), jnp.float32),
        pltpu.VMEM((P2W,), jnp.float32),
        pltpu.SemaphoreType.DMA,
    ],
)


def _final_body(p1_ref, p2_ref, out_ref):
    p1 = p1_ref[...]
    s = jnp.sum(p1, axis=1)
    counts = s[:, F, :]
    counts_safe = jnp.maximum(counts, 1.0)
    mu = s[:, :F, :] / counts_safe[:, None, :]
    musq = jnp.sum(mu * mu, axis=1)
    l_reg = jnp.sum(jnp.sqrt(musq), axis=1) / K

    p2 = p2_ref[...]
    lsum = jnp.sum(p2[:, :, :K], axis=1)
    dmax = jnp.max(p2[:, :, K:], axis=(1, 2))
    l_var = jnp.sum(lsum / counts_safe, axis=1) / K
    nonempty = jnp.sum((counts > 0.0).astype(jnp.float32), axis=1)
    l_var = jnp.where(dmax < 0.0, dmax * dmax * nonempty / K, l_var)

    ci = lax.broadcasted_iota(jnp.int32, (1, B), 1)
    ld_vec = jnp.zeros((1, B), jnp.float32)
    eye_i = lax.broadcasted_iota(jnp.int32, (K, K), 0)
    eye_j = lax.broadcasted_iota(jnp.int32, (K, K), 1)
    off_diag = jnp.where(eye_i == eye_j, 0.0, 1.0)
    for bb in range(B):
        m = mu[bb]
        g = lax.dot_general(m, m, (((0,), (0,)), ((), ())))
        nsq = musq[bb]
        d2 = jnp.maximum(nsq[:, None] + nsq[None, :] - 2.0 * g, 0.0)
        d = jnp.sqrt(d2)
        lm = 2.0 * DELTA_D * off_diag - d
        r = jnp.maximum(lm, 0.0) * off_diag
        ld_b = jnp.sum(r * r) / K / (K - 1)
        ld_vec = ld_vec + ld_b * jnp.where(ci == bb, 1.0, 0.0)

    lv_vec = l_var[None, :]
    lr_vec = l_reg[None, :]
    loss_vec = ALPHA * lv_vec + BETA * ld_vec + GAMMA * lr_vec
    lm_ = jnp.sum(loss_vec) / B
    vm_ = jnp.sum(lv_vec) / B
    dm_ = jnp.sum(ld_vec) / B
    rm_ = jnp.sum(lr_vec) / B
    col = lax.broadcasted_iota(jnp.int32, (8, 128), 1)
    out_ref[...] = (jnp.where(col == 0, lm_, 0.0) + jnp.where(col == 1, vm_, 0.0)
                    + jnp.where(col == 2, dm_, 0.0) + jnp.where(col == 3, rm_, 0.0))


_finalize = pl.pallas_call(
    _final_body,
    out_shape=jax.ShapeDtypeStruct((8, 128), jnp.float32),
)


def kernel(prediction, target):
    pred_flat = prediction.reshape(-1)
    lab_flat = target.astype(jnp.int32).reshape(-1)
    p1, p2 = _fused(pred_flat, lab_flat)
    res = _finalize(p1.reshape(B, TPS, F + 1, K), p2.reshape(B, TPS, P2W))
    return (res[0, 0], res[0, 1], res[0, 2], res[0, 3])


# speedup vs baseline: 64.1294x; 1.3390x over previous
import functools

import jax
import jax.numpy as jnp
from jax import lax
from jax.experimental import pallas as pl
from jax.experimental.pallas import tpu as pltpu
from jax.experimental.pallas import tpu_sc as plsc

B = 4
F = 16
HW = 512 * 512
K = 32
DELTA_V = 0.5
DELTA_D = 1.5
ALPHA, BETA, GAMMA = 1.0, 1.0, 0.001

NC = 2
NS = 16
NW = NC * NS
TPS = NW // B
PPT = HW // TPS
BLK = 2048
NVREG = BLK // 16
NBLK = PPT // BLK
ACC1 = (F + 1) * K
P2W = 48

_mesh = plsc.VectorSubcoreMesh(core_axis_name="c", subcore_axis_name="s")
_sc_params = pltpu.CompilerParams(needs_layout_passes=False)


def _sqrt16(x):
    i = plsc.bitcast(x, jnp.int32)
    z = plsc.bitcast(0x5F3759DF - (i >> 1), jnp.float32)
    z = z * (1.5 - 0.5 * x * z * z)
    z = z * (1.5 - 0.5 * x * z * z)
    z = z * (1.5 - 0.5 * x * z * z)
    return x * z


def _issue_block(pred_hbm, lab_hbm, pbuf, lbuf, sem, b, chunk, bi, phase):
    for f in range(F):
        src = (b * F + f) * HW + chunk * PPT + bi * BLK
        pltpu.async_copy(pred_hbm.at[pl.ds(src, BLK)],
                         pbuf.at[pl.ds((phase * F + f) * BLK, BLK)], sem)
    off = b * HW + chunk * PPT + bi * BLK
    pltpu.async_copy(lab_hbm.at[pl.ds(off, BLK)],
                     lbuf.at[pl.ds(phase * BLK, BLK)], sem)


def _wait_block(pred_hbm, lab_hbm, pbuf, lbuf, sem, phase):
    pltpu.make_async_copy(pred_hbm.at[pl.ds(0, F * BLK)],
                          pbuf.at[pl.ds(phase * F * BLK, F * BLK)], sem).wait()
    pltpu.make_async_copy(lab_hbm.at[pl.ds(0, BLK)],
                          lbuf.at[pl.ds(phase * BLK, BLK)], sem).wait()


def _fused_body(pred_hbm, lab_hbm, p1_hbm, p2_hbm, pbuf, lbuf, acc, p1buf,
                mu, obuf, sem):
    w = lax.axis_index("c") * NS + lax.axis_index("s")
    b = w // TPS
    chunk = w % TPS
    zero = jnp.zeros((16,), jnp.float32)
    ones = jnp.ones((16,), jnp.float32)

    issue = functools.partial(_issue_block, pred_hbm, lab_hbm, pbuf, lbuf,
                              sem, b, chunk)
    wait = functools.partial(_wait_block, pred_hbm, lab_hbm, pbuf, lbuf, sem)

    for i in range(ACC1 // 16):
        acc[pl.ds(i * 16, 16)] = zero

    issue(0, 0)
    issue(1, 1)

    def p1_block(g, carry):
        for phase in range(2):
            bi = g * 2 + phase
            wait(phase)

            def vbody(jj, c2):
                labs = [lbuf[pl.ds(phase * BLK + (jj * 2 + u) * 16, 16)]
                        for u in range(2)]
                valss = [[pbuf[pl.ds((phase * F + f) * BLK + (jj * 2 + u) * 16, 16)]
                          for f in range(F)] for u in range(2)]
                for u in range(2):
                    for f in range(F):
                        plsc.addupdate_scatter(acc, [labs[u] + f * K], valss[u][f])
                    plsc.addupdate_scatter(acc, [labs[u] + F * K], ones)
                return c2

            lax.fori_loop(0, NVREG // 2, vbody, 0)
            issue((bi + 2) % NBLK, phase)
        return carry

    lax.fori_loop(0, NBLK // 2, p1_block, 0)
    wait(0)
    wait(1)

    pltpu.sync_copy(acc, p1_hbm.at[pl.ds(w * ACC1, ACC1)])
    plsc.subcore_barrier()

    pltpu.sync_copy(p1_hbm.at[pl.ds(b * TPS * ACC1, TPS * ACC1)], p1buf)
    invc = []
    for half in range(2):
        c = p1buf[pl.ds(F * K + half * 16, 16)]
        for t in range(1, TPS):
            c = c + p1buf[pl.ds(t * ACC1 + F * K + half * 16, 16)]
        invc.append(1.0 / jnp.maximum(c, 1.0))
    for f in range(F):
        for half in range(2):
            s = p1buf[pl.ds(f * K + half * 16, 16)]
            for t in range(1, TPS):
                s = s + p1buf[pl.ds(t * ACC1 + f * K + half * 16, 16)]
            mu[pl.ds(f * K + half * 16, 16)] = s * invc[half]

    for i in range(P2W // 16):
        obuf[pl.ds(i * 16, 16)] = zero

    issue(0, 0)
    issue(1, 1)

    def p2_block(g, dmax):
        for phase in range(2):
            bi = g * 2 + phase
            wait(phase)

            def vbody(jj, dmx):
                for u in range(2):
                    j = jj * 2 + u
                    lab = lbuf[pl.ds(phase * BLK + j * 16, 16)]
                    a = [None, None, None, None]
                    for half in range(2):
                        fs = range(half * 8, half * 8 + 8)
                        ms = [plsc.load_gather(mu, [lab + f * K]) for f in fs]
                        ps = [pbuf[pl.ds((phase * F + f) * BLK + j * 16, 16)]
                              for f in fs]
                        for i in range(8):
                            diff = ms[i] - ps[i]
                            k = i % 4
                            sq = diff * diff
                            a[k] = sq if a[k] is None else a[k] + sq
                    d2 = (a[0] + a[1]) + (a[2] + a[3])
                    d = _sqrt16(d2) - DELTA_V
                    r = jnp.maximum(d, 0.0)
                    plsc.addupdate_scatter(obuf, [lab], r * r)
                    dmx = jnp.maximum(dmx, d)
                return dmx

            dmax = lax.fori_loop(0, NVREG // 2, vbody, dmax)
            issue((bi + 2) % NBLK, phase)
        return dmax

    dmax = lax.fori_loop(0, NBLK // 2, p2_block,
                         jnp.full((16,), -3e38, jnp.float32))
    wait(0)
    wait(1)

    obuf[pl.ds(32, 16)] = dmax
    pltpu.sync_copy(obuf, p2_hbm.at[pl.ds(w * P2W, P2W)])


_fused = pl.kernel(
    _fused_body,
    out_type=(jax.ShapeDtypeStruct((NW * ACC1,), jnp.float32),
              jax.ShapeDtypeStruct((NW * P2W,), jnp.float32)),
    mesh=_mesh,
    compiler_params=_sc_params,
    scratch_types=[
        pltpu.VMEM((2 * F * BLK,), jnp.float32),
        pltpu.VMEM((2 * BLK,), jnp.int32),
        pltpu.VMEM((ACC1,), jnp.float32),
        pltpu.VMEM((TPS * ACC1,), jnp.float32),
        pltpu.VMEM((F * K,), jnp.float32),
        pltpu.VMEM((P2W,), jnp.float32),
        pltpu.SemaphoreType.DMA,
    ],
)


def _final_body(p1_ref, p2_ref, out_ref):
    p1 = p1_ref[...]
    s = jnp.sum(p1, axis=1)
    counts = s[:, F, :]
    counts_safe = jnp.maximum(counts, 1.0)
    mu = s[:, :F, :] / counts_safe[:, None, :]
    musq = jnp.sum(mu * mu, axis=1)
    l_reg = jnp.sum(jnp.sqrt(musq), axis=1) / K

    p2 = p2_ref[...]
    lsum = jnp.sum(p2[:, :, :K], axis=1)
    dmax = jnp.max(p2[:, :, K:], axis=(1, 2))
    l_var = jnp.sum(lsum / counts_safe, axis=1) / K
    nonempty = jnp.sum((counts > 0.0).astype(jnp.float32), axis=1)
    l_var = jnp.where(dmax < 0.0, dmax * dmax * nonempty / K, l_var)

    ci = lax.broadcasted_iota(jnp.int32, (1, B), 1)
    ld_vec = jnp.zeros((1, B), jnp.float32)
    eye_i = lax.broadcasted_iota(jnp.int32, (K, K), 0)
    eye_j = lax.broadcasted_iota(jnp.int32, (K, K), 1)
    off_diag = jnp.where(eye_i == eye_j, 0.0, 1.0)
    for bb in range(B):
        m = mu[bb]
        g = lax.dot_general(m, m, (((0,), (0,)), ((), ())))
        nsq = musq[bb]
        d2 = jnp.maximum(nsq[:, None] + nsq[None, :] - 2.0 * g, 0.0)
        d = jnp.sqrt(d2)
        lm = 2.0 * DELTA_D * off_diag - d
        r = jnp.maximum(lm, 0.0) * off_diag
        ld_b = jnp.sum(r * r) / K / (K - 1)
        ld_vec = ld_vec + ld_b * jnp.where(ci == bb, 1.0, 0.0)

    lv_vec = l_var[None, :]
    lr_vec = l_reg[None, :]
    loss_vec = ALPHA * lv_vec + BETA * ld_vec + GAMMA * lr_vec
    lm_ = jnp.sum(loss_vec) / B
    vm_ = jnp.sum(lv_vec) / B
    dm_ = jnp.sum(ld_vec) / B
    rm_ = jnp.sum(lr_vec) / B
    col = lax.broadcasted_iota(jnp.int32, (8, 128), 1)
    out_ref[...] = (jnp.where(col == 0, lm_, 0.0) + jnp.where(col == 1, vm_, 0.0)
                    + jnp.where(col == 2, dm_, 0.0) + jnp.where(col == 3, rm_, 0.0))


_finalize = pl.pallas_call(
    _final_body,
    out_shape=jax.ShapeDtypeStruct((8, 128), jnp.float32),
)


def kernel(prediction, target):
    pred_flat = prediction.reshape(-1)
    lab_flat = target.astype(jnp.int32).reshape(-1)
    p1, p2 = _fused(pred_flat, lab_flat)
    res = _finalize(p1.reshape(B, TPS, F + 1, K), p2.reshape(B, TPS, P2W))
    return (res[0, 0], res[0, 1], res[0, 2], res[0, 3])
